# baseline (speedup 1.0000x reference)
import functools

import jax
import jax.numpy as jnp
from jax import lax
from jax.experimental import pallas as pl
from jax.experimental.pallas import tpu as pltpu

BV = 256
BO = 512


def _pool_body(nv_ref, x_ref, o_ref, *, bv, max_views, spb):
    bi = pl.program_id(0)
    j = pl.program_id(1)

    @pl.when(j == 0)
    def _init():
        o_ref[...] = jnp.full_like(o_ref, -jnp.inf)

    for l in range(spb):
        nv = jnp.minimum(nv_ref[bi * spb + l], max_views)
        jmax = (nv + bv - 1) // bv - 1

        @pl.when(j <= jmax)
        def _update(l=l, nv=nv):
            row = j * bv + lax.broadcasted_iota(jnp.int32, (bv, 1), 0)
            blk = jnp.where(row < nv, x_ref[l], -jnp.inf)
            part = blk[0:8]
            for r in range(1, bv // 8):
                part = jnp.maximum(part, blk[r * 8:(r + 1) * 8])
            o_ref[l] = jnp.maximum(o_ref[l],
                                   jnp.max(part, axis=0, keepdims=True))


def _linear_body(k_ref, w_ref, bias_ref, o_ref):
    out = lax.dot_general(
        k_ref[...], w_ref[...],
        dimension_numbers=(((1,), (1,)), ((), ())),
        preferred_element_type=jnp.float32,
    )
    o_ref[...] = out + bias_ref[...]


def kernel(batch_size, max_num_views, num_views, x, W, b):
    B, V, D = x.shape
    O = W.shape[0]

    SPB = 4

    pool = pl.pallas_call(
        functools.partial(_pool_body, bv=BV, max_views=V, spb=SPB),
        grid_spec=pltpu.PrefetchScalarGridSpec(
            num_scalar_prefetch=1,
            grid=(B // SPB, V // BV),
            in_specs=[pl.BlockSpec((SPB, BV, D),
                                   lambda bi, j, nv_ref: (bi, j, 0))],
            out_specs=pl.BlockSpec((SPB, 1, D),
                                   lambda bi, j, nv_ref: (bi, 0, 0)),
        ),
        out_shape=jax.ShapeDtypeStruct((B, 1, D), jnp.float32),
        compiler_params=pltpu.CompilerParams(
            dimension_semantics=("arbitrary", "arbitrary"),
        ),
    )
    k = pool(num_views.astype(jnp.int32), x).reshape(B, D)

    bias = b.reshape(1, O)
    linear = pl.pallas_call(
        _linear_body,
        grid=(O // BO,),
        in_specs=[
            pl.BlockSpec((B, D), lambda o: (0, 0)),
            pl.BlockSpec((BO, D), lambda o: (o, 0)),
            pl.BlockSpec((1, BO), lambda o: (0, o)),
        ],
        out_specs=pl.BlockSpec((B, BO), lambda o: (0, o)),
        out_shape=jax.ShapeDtypeStruct((B, O), jnp.float32),
        compiler_params=pltpu.CompilerParams(
            dimension_semantics=("arbitrary",),
        ),
    )
    logits = linear(k, W, bias)
    return (logits, k)
